# baseline (speedup 1.0000x reference)
import functools

import jax
import jax.numpy as jnp
from jax import lax
from jax.experimental import pallas as pl
from jax.experimental.pallas import tpu as pltpu
from jax.experimental.pallas import tpu_sc as plsc

N_TILES = 32
CHUNK = 128
ZROWS = 136
NBUF = 8
HALVES = 2


def _spmm_sc(row3, col3, val3, x):
    n, f = x.shape
    cpt = row3.shape[1]
    hpt = cpt // HALVES
    rpt = n // 16
    mesh = plsc.VectorSubcoreMesh(core_axis_name="c", subcore_axis_name="s")

    @functools.partial(
        pl.kernel,
        out_type=jax.ShapeDtypeStruct((2, n, f), jnp.float32),
        mesh=mesh,
        scratch_types=[
            pltpu.VMEM((hpt, CHUNK), jnp.int32),
            pltpu.VMEM((hpt, CHUNK), jnp.int32),
            pltpu.VMEM((hpt, CHUNK), jnp.float32),
            pltpu.VMEM((NBUF, CHUNK, f), jnp.float32),
            pltpu.VMEM((ZROWS, f), jnp.float32),
            pltpu.VMEM_SHARED((n, f), jnp.float32),
        ]
        + [pltpu.SemaphoreType.DMA] * NBUF,
        compiler_params=pltpu.CompilerParams(use_tc_tiling_on_sc=False),
    )
    def spmm_kernel(row_hbm, col_hbm, val_hbm, x_hbm, out_hbm,
                    row_v, col_v, val_v, rows_v, zero_v, acc, *sems):
        c = lax.axis_index("c")
        s = lax.axis_index("s")
        w = c * 16 + s

        @pl.loop(0, ZROWS)
        def _(r):
            zero_v[r, :] = jnp.zeros((f,), jnp.float32)

        base = s * rpt

        @pl.loop(0, rpt // ZROWS)
        def _(i):
            pltpu.sync_copy(zero_v, acc.at[pl.ds(base + i * ZROWS, ZROWS)])

        plsc.subcore_barrier()

        for h in range(HALVES):
            pltpu.sync_copy(row_hbm.at[w, pl.ds(h * hpt, hpt)], row_v)
            pltpu.sync_copy(col_hbm.at[w, pl.ds(h * hpt, hpt)], col_v)
            pltpu.sync_copy(val_hbm.at[w, pl.ds(h * hpt, hpt)], val_v)

            for b in range(NBUF):
                pltpu.async_copy(x_hbm.at[col_v.at[b]], rows_v.at[b],
                                 sems[b])

            @pl.loop(0, hpt // NBUF)
            def _(i):
                for b in range(NBUF):
                    j = i * NBUF + b
                    pltpu.make_async_copy(
                        x_hbm.at[col_v.at[j]], rows_v.at[b], sems[b]).wait()

                    @pl.loop(0, CHUNK // 16)
                    def _(k):
                        vv = val_v[j, pl.ds(k * 16, 16)]
                        for u in range(16):
                            e0 = k * 16 + u
                            rows_v[b, e0, :] = rows_v[b, e0, :] * vv[u]

                    pltpu.sync_copy(rows_v.at[b], acc.at[row_v.at[j]],
                                    add=True)

                    nxt = j + NBUF

                    @pl.when(nxt < hpt)
                    def _():
                        pltpu.async_copy(
                            x_hbm.at[col_v.at[nxt]], rows_v.at[b], sems[b])

        plsc.subcore_barrier()

        pltpu.sync_copy(acc.at[pl.ds(base, rpt)],
                        out_hbm.at[c, pl.ds(base, rpt)])

    return spmm_kernel(row3, col3, val3, x)


def _spmm1_sc(row3, col3, val3, xv):
    n1 = xv.shape[0]
    cpt = row3.shape[1]
    hpt = cpt // HALVES
    rpt = n1 // 16
    mesh = plsc.VectorSubcoreMesh(core_axis_name="c", subcore_axis_name="s")

    @functools.partial(
        pl.kernel,
        out_type=jax.ShapeDtypeStruct((2, n1), jnp.float32),
        mesh=mesh,
        scratch_types=[
            pltpu.VMEM((hpt, CHUNK), jnp.int32),
            pltpu.VMEM((hpt, CHUNK), jnp.int32),
            pltpu.VMEM((hpt, CHUNK), jnp.float32),
            pltpu.VMEM((NBUF, CHUNK), jnp.float32),
            pltpu.VMEM((rpt,), jnp.float32),
            pltpu.VMEM_SHARED((n1,), jnp.float32),
        ]
        + [pltpu.SemaphoreType.DMA] * NBUF,
        compiler_params=pltpu.CompilerParams(use_tc_tiling_on_sc=False),
    )
    def spmm1_kernel(row_hbm, col_hbm, val_hbm, x_hbm, out_hbm,
                     row_v, col_v, val_v, rows_v, zero_v, acc, *sems):
        c = lax.axis_index("c")
        s = lax.axis_index("s")
        w = c * 16 + s

        @pl.loop(0, rpt // 16)
        def _(r):
            zero_v[pl.ds(r * 16, 16)] = jnp.zeros((16,), jnp.float32)

        base = s * rpt
        pltpu.sync_copy(zero_v, acc.at[pl.ds(base, rpt)])
        plsc.subcore_barrier()

        for h in range(HALVES):
            pltpu.sync_copy(row_hbm.at[w, pl.ds(h * hpt, hpt)], row_v)
            pltpu.sync_copy(col_hbm.at[w, pl.ds(h * hpt, hpt)], col_v)
            pltpu.sync_copy(val_hbm.at[w, pl.ds(h * hpt, hpt)], val_v)

            for b in range(NBUF):
                pltpu.async_copy(x_hbm.at[col_v.at[b]], rows_v.at[b],
                                 sems[b])

            @pl.loop(0, hpt // NBUF)
            def _(i):
                for b in range(NBUF):
                    j = i * NBUF + b
                    pltpu.make_async_copy(
                        x_hbm.at[col_v.at[j]], rows_v.at[b], sems[b]).wait()

                    @pl.loop(0, CHUNK // 16)
                    def _(k):
                        sl = pl.ds(k * 16, 16)
                        rows_v[b, sl] = rows_v[b, sl] * val_v[j, sl]

                    pltpu.sync_copy(rows_v.at[b], acc.at[row_v.at[j]],
                                    add=True)

                    nxt = j + NBUF

                    @pl.when(nxt < hpt)
                    def _():
                        pltpu.async_copy(
                            x_hbm.at[col_v.at[nxt]], rows_v.at[b], sems[b])

        plsc.subcore_barrier()

        pltpu.sync_copy(acc.at[pl.ds(base, rpt)],
                        out_hbm.at[c, pl.ds(base, rpt)])

    return spmm1_kernel(row3, col3, val3, xv)


def _tc_map(body, out_feat, *arrays, block_rows=6256):
    n = arrays[0].shape[0]
    grid = (n // block_rows,)
    in_specs = []
    for a in arrays:
        if a.shape[0] == n:
            blk = (block_rows,) + a.shape[1:]
            in_specs.append(pl.BlockSpec(blk, lambda i, _nd=a.ndim: (i,) + (0,) * (_nd - 1)))
        else:
            in_specs.append(pl.BlockSpec(a.shape, lambda i, _nd=a.ndim: (0,) * _nd))
    out = jax.ShapeDtypeStruct((n, out_feat), jnp.float32)

    def wrapped(*refs):
        ins = refs[:-1]
        o = refs[-1]
        o[...] = body(*[r[...] for r in ins])

    return pl.pallas_call(
        wrapped,
        grid=grid,
        in_specs=in_specs,
        out_specs=pl.BlockSpec((block_rows, out_feat), lambda i: (i, 0)),
        out_shape=out,
    )(*arrays)


def _dot(a, b):
    return jax.lax.dot_general(a, b, (((1,), (0,)), ((), ())),
                               preferred_element_type=jnp.float32)


def kernel(laplacian_indices, laplacian_values, x,
           W1, W1_skip, b1, W2, b2, W3, b3, Wf, bf):
    n_orig = x.shape[0]
    e = laplacian_values.shape[0]

    n = -(-n_orig // (16 * ZROWS)) * (16 * ZROWS)
    x = jnp.pad(x, ((0, n - n_orig), (0, 0)))

    egran = CHUNK * NBUF * HALVES
    per_tile = -(-e // (N_TILES * egran)) * egran
    e_pad = per_tile * N_TILES
    cpt = per_tile // CHUNK
    pad = e_pad - e
    row = jnp.pad(laplacian_indices[0], (0, pad)).reshape(N_TILES, cpt, CHUNK)
    col = jnp.pad(laplacian_indices[1], (0, pad)).reshape(N_TILES, cpt, CHUNK)
    val = jnp.pad(laplacian_values, (0, pad)).reshape(N_TILES, cpt, CHUNK)

    spmm = lambda h: _spmm_sc(row, col, val, h)

    n1 = -(-n_orig // 256) * 256
    x1 = jnp.pad(x[:, 0], (0, n1 - n))
    p = _spmm1_sc(row, col, val, x1)
    pr = p.reshape(2, n1 // 128, 128)
    t1 = _tc_map(lambda a, b: a + b, 128, pr[0], pr[1],
                 block_rows=n1 // 128).reshape(n1)
    q = _spmm1_sc(row, col, val, t1)
    h1 = _tc_map(
        lambda xb, t1b, q0b, q1b, W, Ws, bb: jax.nn.relu(
            xb * (W[0][0] + Ws[0] - W[2][0])[None, :]
            + t1b * W[1][0][None, :]
            + (q0b + q1b) * (2.0 * W[2][0])[None, :]
            + bb[0][None, :]),
        16, x, t1[:n, None], q[0, :n, None], q[1, :n, None],
        W1, W1_skip, b1.reshape(1, -1))

    p = spmm(h1)
    T1 = _tc_map(lambda p0, p1: p0 + p1, 16, p[0], p[1])
    q = spmm(T1)
    h2 = _tc_map(
        lambda hb, t1, W, bb, q0, q1: jax.nn.relu(
            _dot(hb, W[0] - W[2]) + _dot(t1, W[1])
            + _dot(q0 + q1, 2.0 * W[2]) + bb[0][None, :]),
        64, h1, T1, W2, b2.reshape(1, -1), q[0], q[1])

    c2 = _tc_map(lambda hb, W: _dot(hb, 2.0 * W[2]), 16, h2, W3)
    p = spmm(c2)
    mid = _tc_map(lambda hb, W, p0, p1: _dot(hb, W[1]) + p0 + p1,
                  16, h2, W3, p[0], p[1])
    q = spmm(mid)
    out = _tc_map(
        lambda hb, W, bb, q0, q1, Wff, bff: _dot(
            jax.nn.relu(_dot(hb, W[0] - W[2]) + q0 + q1 + bb[0][None, :]),
            Wff[0]) + bff[0][None, :],
        1, h2, W3, b3.reshape(1, -1), q[0], q[1], Wf, bf.reshape(1, -1))

    return out[:n_orig]


# speedup vs baseline: 1.6352x; 1.6352x over previous
import functools

import jax
import jax.numpy as jnp
from jax import lax
from jax.experimental import pallas as pl
from jax.experimental.pallas import tpu as pltpu
from jax.experimental.pallas import tpu_sc as plsc

N_TILES = 32
CHUNK = 128
ZROWS = 136
NBUF = 4
HALVES = 2


def _spmm_sc(row3, col3, val3, x):
    n, f = x.shape
    cpt = row3.shape[1]
    hpt = cpt // HALVES
    rpt = n // 16
    mesh = plsc.VectorSubcoreMesh(core_axis_name="c", subcore_axis_name="s")

    @functools.partial(
        pl.kernel,
        out_type=jax.ShapeDtypeStruct((2, n, f), jnp.float32),
        mesh=mesh,
        scratch_types=[
            pltpu.VMEM((hpt, CHUNK), jnp.int32),
            pltpu.VMEM((hpt, CHUNK), jnp.int32),
            pltpu.VMEM((hpt, CHUNK), jnp.float32),
            pltpu.VMEM((NBUF, CHUNK, f), jnp.float32),
            pltpu.VMEM((ZROWS, f), jnp.float32),
            pltpu.VMEM_SHARED((n, f), jnp.float32),
        ]
        + [pltpu.SemaphoreType.DMA] * NBUF,
        compiler_params=pltpu.CompilerParams(use_tc_tiling_on_sc=False),
    )
    def spmm_kernel(row_hbm, col_hbm, val_hbm, x_hbm, out_hbm,
                    row_v, col_v, val_v, rows_v, zero_v, acc, *sems):
        c = lax.axis_index("c")
        s = lax.axis_index("s")
        w = c * 16 + s

        @pl.loop(0, ZROWS)
        def _(r):
            zero_v[r, :] = jnp.zeros((f,), jnp.float32)

        base = s * rpt

        @pl.loop(0, rpt // ZROWS)
        def _(i):
            pltpu.sync_copy(zero_v, acc.at[pl.ds(base + i * ZROWS, ZROWS)])

        plsc.subcore_barrier()

        for h in range(HALVES):
            pltpu.sync_copy(row_hbm.at[w, pl.ds(h * hpt, hpt)], row_v)
            pltpu.sync_copy(col_hbm.at[w, pl.ds(h * hpt, hpt)], col_v)
            pltpu.sync_copy(val_hbm.at[w, pl.ds(h * hpt, hpt)], val_v)

            for b in range(NBUF):
                pltpu.async_copy(x_hbm.at[col_v.at[b]], rows_v.at[b],
                                 sems[b])

            @pl.loop(0, hpt // NBUF)
            def _(i):
                for b in range(NBUF):
                    j = i * NBUF + b
                    pltpu.make_async_copy(
                        x_hbm.at[col_v.at[j]], rows_v.at[b], sems[b]).wait()

                    @pl.loop(0, CHUNK // 16)
                    def _(k):
                        vv = val_v[j, pl.ds(k * 16, 16)]
                        for u in range(16):
                            e0 = k * 16 + u
                            rows_v[b, e0, :] = rows_v[b, e0, :] * vv[u]

                    pltpu.sync_copy(rows_v.at[b], acc.at[row_v.at[j]],
                                    add=True)

                    nxt = j + NBUF

                    @pl.when(nxt < hpt)
                    def _():
                        pltpu.async_copy(
                            x_hbm.at[col_v.at[nxt]], rows_v.at[b], sems[b])

        plsc.subcore_barrier()

        pltpu.sync_copy(acc.at[pl.ds(base, rpt)],
                        out_hbm.at[c, pl.ds(base, rpt)])

    return spmm_kernel(row3, col3, val3, x)


def _spmm1_sc(row3, col3, val3, xv):
    n1 = xv.shape[0]
    cpt = row3.shape[1]
    hpt = cpt // HALVES
    rpt = n1 // 16
    mesh = plsc.VectorSubcoreMesh(core_axis_name="c", subcore_axis_name="s")

    @functools.partial(
        pl.kernel,
        out_type=jax.ShapeDtypeStruct((2, n1), jnp.float32),
        mesh=mesh,
        scratch_types=[
            pltpu.VMEM((hpt, CHUNK), jnp.int32),
            pltpu.VMEM((hpt, CHUNK), jnp.int32),
            pltpu.VMEM((hpt, CHUNK), jnp.float32),
            pltpu.VMEM((NBUF, CHUNK), jnp.float32),
            pltpu.VMEM((rpt,), jnp.float32),
            pltpu.VMEM_SHARED((n1,), jnp.float32),
        ]
        + [pltpu.SemaphoreType.DMA] * NBUF,
        compiler_params=pltpu.CompilerParams(use_tc_tiling_on_sc=False),
    )
    def spmm1_kernel(row_hbm, col_hbm, val_hbm, x_hbm, out_hbm,
                     row_v, col_v, val_v, rows_v, zero_v, acc, *sems):
        c = lax.axis_index("c")
        s = lax.axis_index("s")
        w = c * 16 + s

        @pl.loop(0, rpt // 16)
        def _(r):
            zero_v[pl.ds(r * 16, 16)] = jnp.zeros((16,), jnp.float32)

        base = s * rpt
        pltpu.sync_copy(zero_v, acc.at[pl.ds(base, rpt)])
        plsc.subcore_barrier()

        for h in range(HALVES):
            pltpu.sync_copy(row_hbm.at[w, pl.ds(h * hpt, hpt)], row_v)
            pltpu.sync_copy(col_hbm.at[w, pl.ds(h * hpt, hpt)], col_v)
            pltpu.sync_copy(val_hbm.at[w, pl.ds(h * hpt, hpt)], val_v)

            for b in range(NBUF):
                pltpu.async_copy(x_hbm.at[col_v.at[b]], rows_v.at[b],
                                 sems[b])

            @pl.loop(0, hpt // NBUF)
            def _(i):
                for b in range(NBUF):
                    j = i * NBUF + b
                    pltpu.make_async_copy(
                        x_hbm.at[col_v.at[j]], rows_v.at[b], sems[b]).wait()

                    @pl.loop(0, CHUNK // 16)
                    def _(k):
                        sl = pl.ds(k * 16, 16)
                        rows_v[b, sl] = rows_v[b, sl] * val_v[j, sl]

                    pltpu.sync_copy(rows_v.at[b], acc.at[row_v.at[j]],
                                    add=True)

                    nxt = j + NBUF

                    @pl.when(nxt < hpt)
                    def _():
                        pltpu.async_copy(
                            x_hbm.at[col_v.at[nxt]], rows_v.at[b], sems[b])

        plsc.subcore_barrier()

        pltpu.sync_copy(acc.at[pl.ds(base, rpt)],
                        out_hbm.at[c, pl.ds(base, rpt)])

    return spmm1_kernel(row3, col3, val3, xv)


def _tc_map(body, out_feat, *arrays, block_rows=6256):
    n = arrays[0].shape[0]
    grid = (n // block_rows,)
    in_specs = []
    for a in arrays:
        if a.shape[0] == n:
            blk = (block_rows,) + a.shape[1:]
            in_specs.append(pl.BlockSpec(blk, lambda i, _nd=a.ndim: (i,) + (0,) * (_nd - 1)))
        else:
            in_specs.append(pl.BlockSpec(a.shape, lambda i, _nd=a.ndim: (0,) * _nd))
    out = jax.ShapeDtypeStruct((n, out_feat), jnp.float32)

    def wrapped(*refs):
        ins = refs[:-1]
        o = refs[-1]
        o[...] = body(*[r[...] for r in ins])

    return pl.pallas_call(
        wrapped,
        grid=grid,
        in_specs=in_specs,
        out_specs=pl.BlockSpec((block_rows, out_feat), lambda i: (i, 0)),
        out_shape=out,
    )(*arrays)


def _dot(a, b):
    return jax.lax.dot_general(a, b, (((1,), (0,)), ((), ())),
                               preferred_element_type=jnp.float32)


def kernel(laplacian_indices, laplacian_values, x,
           W1, W1_skip, b1, W2, b2, W3, b3, Wf, bf):
    n_orig = x.shape[0]
    e = laplacian_values.shape[0]

    n = -(-n_orig // (16 * ZROWS)) * (16 * ZROWS)
    x = jnp.pad(x, ((0, n - n_orig), (0, 0)))

    egran = CHUNK * NBUF * HALVES
    per_tile = -(-e // (N_TILES * egran)) * egran
    e_pad = per_tile * N_TILES
    cpt = per_tile // CHUNK
    pad = e_pad - e
    row = jnp.pad(laplacian_indices[0], (0, pad)).reshape(N_TILES, cpt, CHUNK)
    col = jnp.pad(laplacian_indices[1], (0, pad)).reshape(N_TILES, cpt, CHUNK)
    val = jnp.pad(laplacian_values, (0, pad)).reshape(N_TILES, cpt, CHUNK)

    spmm = lambda h: _spmm_sc(row, col, val, h)

    n1 = -(-n_orig // 256) * 256
    x1 = jnp.pad(x[:, 0], (0, n1 - n))
    p = _spmm1_sc(row, col, val, x1)
    pr = p.reshape(2, n1 // 128, 128)
    t1 = _tc_map(lambda a, b: a + b, 128, pr[0], pr[1],
                 block_rows=n1 // 128).reshape(n1)
    q = _spmm1_sc(row, col, val, t1)
    h1 = _tc_map(
        lambda xb, t1b, q0b, q1b, W, Ws, bb: jax.nn.relu(
            xb * (W[0][0] + Ws[0] - W[2][0])[None, :]
            + t1b * W[1][0][None, :]
            + (q0b + q1b) * (2.0 * W[2][0])[None, :]
            + bb[0][None, :]),
        16, x, t1[:n, None], q[0, :n, None], q[1, :n, None],
        W1, W1_skip, b1.reshape(1, -1))

    p = spmm(h1)
    T1 = _tc_map(lambda p0, p1: p0 + p1, 16, p[0], p[1])
    q = spmm(T1)
    h2 = _tc_map(
        lambda hb, t1, W, bb, q0, q1: jax.nn.relu(
            _dot(hb, W[0] - W[2]) + _dot(t1, W[1])
            + _dot(q0 + q1, 2.0 * W[2]) + bb[0][None, :]),
        64, h1, T1, W2, b2.reshape(1, -1), q[0], q[1])

    c2 = _tc_map(lambda hb, W: _dot(hb, 2.0 * W[2]), 16, h2, W3)
    p = spmm(c2)
    mid = _tc_map(lambda hb, W, p0, p1: _dot(hb, W[1]) + p0 + p1,
                  16, h2, W3, p[0], p[1])
    q = spmm(mid)
    out = _tc_map(
        lambda hb, W, bb, q0, q1, Wff, bff: _dot(
            jax.nn.relu(_dot(hb, W[0] - W[2]) + q0 + q1 + bb[0][None, :]),
            Wff[0]) + bff[0][None, :],
        1, h2, W3, b3.reshape(1, -1), q[0], q[1], Wf, bf.reshape(1, -1))

    return out[:n_orig]


# speedup vs baseline: 2.1566x; 1.3189x over previous
import functools

import jax
import jax.numpy as jnp
from jax import lax
from jax.experimental import pallas as pl
from jax.experimental.pallas import tpu as pltpu
from jax.experimental.pallas import tpu_sc as plsc

N_TILES = 32
CHUNK = 128
ZROWS = 136
NBUF = 4
HALVES = 2


def _split_chunks(C):
    HPMAX = 125
    n0 = int(round(0.69 * C / 16.0))
    P0 = max(1, -(-n0 // HPMAX))
    n0 -= n0 % P0
    C0 = 16 * n0
    n1 = -(-(C - C0) // 16)
    P1 = max(1, -(-n1 // HPMAX))
    n1 += (-n1) % P1
    return n0, P0, n1, P1, C0 + 16 * n1


def _spmm_sc(row2, col2, val2, split, x):
    n, f = x.shape
    n0, P0, n1, P1, _ = split
    slab = max(n0 // P0, n1 // P1)
    rpt = n // 16
    mesh = plsc.VectorSubcoreMesh(core_axis_name="c", subcore_axis_name="s")

    @functools.partial(
        pl.kernel,
        out_type=jax.ShapeDtypeStruct((2, n, f), jnp.float32),
        mesh=mesh,
        scratch_types=[
            pltpu.VMEM((slab, CHUNK), jnp.int32),
            pltpu.VMEM((slab, CHUNK), jnp.int32),
            pltpu.VMEM((slab, CHUNK), jnp.float32),
            pltpu.VMEM((NBUF, CHUNK, f), jnp.float32),
            pltpu.VMEM((ZROWS, f), jnp.float32),
            pltpu.VMEM_SHARED((n, f), jnp.float32),
        ]
        + [pltpu.SemaphoreType.DMA] * NBUF,
        compiler_params=pltpu.CompilerParams(use_tc_tiling_on_sc=False),
    )
    def spmm_kernel(row_hbm, col_hbm, val_hbm, x_hbm, out_hbm,
                    row_v, col_v, val_v, rows_v, zero_v, acc, *sems):
        c = lax.axis_index("c")
        s = lax.axis_index("s")

        @pl.loop(0, ZROWS)
        def _(r):
            zero_v[r, :] = jnp.zeros((f,), jnp.float32)

        base = s * rpt

        @pl.loop(0, rpt // ZROWS)
        def _(i):
            pltpu.sync_copy(zero_v, acc.at[pl.ds(base + i * ZROWS, ZROWS)])

        plsc.subcore_barrier()

        def run_pass(cstart, hp):
            pltpu.sync_copy(row_hbm.at[pl.ds(cstart, hp)],
                            row_v.at[pl.ds(0, hp)])
            pltpu.sync_copy(col_hbm.at[pl.ds(cstart, hp)],
                            col_v.at[pl.ds(0, hp)])
            pltpu.sync_copy(val_hbm.at[pl.ds(cstart, hp)],
                            val_v.at[pl.ds(0, hp)])

            for b in range(NBUF):
                pltpu.async_copy(x_hbm.at[col_v.at[b]], rows_v.at[b],
                                 sems[b])

            def step(j, b):
                pltpu.make_async_copy(
                    x_hbm.at[col_v.at[j]], rows_v.at[b], sems[b]).wait()

                @pl.loop(0, CHUNK // 16)
                def _(k):
                    vv = val_v[j, pl.ds(k * 16, 16)]
                    for u in range(16):
                        e0 = k * 16 + u
                        rows_v[b, e0, :] = rows_v[b, e0, :] * vv[u]

                pltpu.sync_copy(rows_v.at[b], acc.at[row_v.at[j]],
                                add=True)

            @pl.loop(0, hp // NBUF)
            def _(i):
                for b in range(NBUF):
                    j = i * NBUF + b
                    step(j, b)
                    nxt = j + NBUF

                    @pl.when(nxt < hp)
                    def _():
                        pltpu.async_copy(
                            x_hbm.at[col_v.at[nxt]], rows_v.at[b], sems[b])

            for b in range(hp % NBUF):
                step((hp // NBUF) * NBUF + b, b)

        @pl.when(c == 0)
        def _():
            for p in range(P0):
                run_pass(s * n0 + p * (n0 // P0), n0 // P0)

        @pl.when(c == 1)
        def _():
            for p in range(P1):
                run_pass(16 * n0 + s * n1 + p * (n1 // P1), n1 // P1)

        plsc.subcore_barrier()

        pltpu.sync_copy(acc.at[pl.ds(base, rpt)],
                        out_hbm.at[c, pl.ds(base, rpt)])

    return spmm_kernel(row2, col2, val2, x)


def _spmm1_sc(row2, col2, val2, split, xv):
    n1 = xv.shape[0]
    n0, P0, nc1, P1, _ = split
    slab = max(n0 // P0, nc1 // P1)
    rpt = n1 // 16
    mesh = plsc.VectorSubcoreMesh(core_axis_name="c", subcore_axis_name="s")

    @functools.partial(
        pl.kernel,
        out_type=jax.ShapeDtypeStruct((2, n1), jnp.float32),
        mesh=mesh,
        scratch_types=[
            pltpu.VMEM((slab, CHUNK), jnp.int32),
            pltpu.VMEM((slab, CHUNK), jnp.int32),
            pltpu.VMEM((slab, CHUNK), jnp.float32),
            pltpu.VMEM((NBUF, CHUNK), jnp.float32),
            pltpu.VMEM((rpt,), jnp.float32),
            pltpu.VMEM_SHARED((n1,), jnp.float32),
        ]
        + [pltpu.SemaphoreType.DMA] * NBUF,
        compiler_params=pltpu.CompilerParams(use_tc_tiling_on_sc=False),
    )
    def spmm1_kernel(row_hbm, col_hbm, val_hbm, x_hbm, out_hbm,
                     row_v, col_v, val_v, rows_v, zero_v, acc, *sems):
        c = lax.axis_index("c")
        s = lax.axis_index("s")

        @pl.loop(0, rpt // 16)
        def _(r):
            zero_v[pl.ds(r * 16, 16)] = jnp.zeros((16,), jnp.float32)

        base = s * rpt
        pltpu.sync_copy(zero_v, acc.at[pl.ds(base, rpt)])
        plsc.subcore_barrier()

        def run_pass(cstart, hp):
            pltpu.sync_copy(row_hbm.at[pl.ds(cstart, hp)],
                            row_v.at[pl.ds(0, hp)])
            pltpu.sync_copy(col_hbm.at[pl.ds(cstart, hp)],
                            col_v.at[pl.ds(0, hp)])
            pltpu.sync_copy(val_hbm.at[pl.ds(cstart, hp)],
                            val_v.at[pl.ds(0, hp)])

            for b in range(NBUF):
                pltpu.async_copy(x_hbm.at[col_v.at[b]], rows_v.at[b],
                                 sems[b])

            def step(j, b):
                pltpu.make_async_copy(
                    x_hbm.at[col_v.at[j]], rows_v.at[b], sems[b]).wait()

                @pl.loop(0, CHUNK // 16)
                def _(k):
                    sl = pl.ds(k * 16, 16)
                    rows_v[b, sl] = rows_v[b, sl] * val_v[j, sl]

                pltpu.sync_copy(rows_v.at[b], acc.at[row_v.at[j]],
                                add=True)

            @pl.loop(0, hp // NBUF)
            def _(i):
                for b in range(NBUF):
                    j = i * NBUF + b
                    step(j, b)
                    nxt = j + NBUF

                    @pl.when(nxt < hp)
                    def _():
                        pltpu.async_copy(
                            x_hbm.at[col_v.at[nxt]], rows_v.at[b], sems[b])

            for b in range(hp % NBUF):
                step((hp // NBUF) * NBUF + b, b)

        @pl.when(c == 0)
        def _():
            for p in range(P0):
                run_pass(s * n0 + p * (n0 // P0), n0 // P0)

        @pl.when(c == 1)
        def _():
            for p in range(P1):
                run_pass(16 * n0 + s * nc1 + p * (nc1 // P1), nc1 // P1)

        plsc.subcore_barrier()

        pltpu.sync_copy(acc.at[pl.ds(base, rpt)],
                        out_hbm.at[c, pl.ds(base, rpt)])

    return spmm1_kernel(row2, col2, val2, xv)


def _tc_map(body, out_feat, *arrays, block_rows=6256):
    n = arrays[0].shape[0]
    grid = (n // block_rows,)
    in_specs = []
    for a in arrays:
        if a.shape[0] == n:
            blk = (block_rows,) + a.shape[1:]
            in_specs.append(pl.BlockSpec(blk, lambda i, _nd=a.ndim: (i,) + (0,) * (_nd - 1)))
        else:
            in_specs.append(pl.BlockSpec(a.shape, lambda i, _nd=a.ndim: (0,) * _nd))
    out = jax.ShapeDtypeStruct((n, out_feat), jnp.float32)

    def wrapped(*refs):
        ins = refs[:-1]
        o = refs[-1]
        o[...] = body(*[r[...] for r in ins])

    return pl.pallas_call(
        wrapped,
        grid=grid,
        in_specs=in_specs,
        out_specs=pl.BlockSpec((block_rows, out_feat), lambda i: (i, 0)),
        out_shape=out,
    )(*arrays)


def _dot(a, b):
    return jax.lax.dot_general(a, b, (((1,), (0,)), ((), ())),
                               preferred_element_type=jnp.float32)


def kernel(laplacian_indices, laplacian_values, x,
           W1, W1_skip, b1, W2, b2, W3, b3, Wf, bf):
    n_orig = x.shape[0]
    e = laplacian_values.shape[0]

    n = -(-n_orig // (16 * ZROWS)) * (16 * ZROWS)
    x = jnp.pad(x, ((0, n - n_orig), (0, 0)))

    split = _split_chunks(-(-e // CHUNK))
    e_pad = split[4] * CHUNK
    pad = e_pad - e
    row = jnp.pad(laplacian_indices[0], (0, pad)).reshape(-1, CHUNK)
    col = jnp.pad(laplacian_indices[1], (0, pad)).reshape(-1, CHUNK)
    val = jnp.pad(laplacian_values, (0, pad)).reshape(-1, CHUNK)

    spmm = lambda h: _spmm_sc(row, col, val, split, h)

    n1 = -(-n_orig // 256) * 256
    x1 = jnp.pad(x[:, 0], (0, n1 - n))
    p = _spmm1_sc(row, col, val, split, x1)
    pr = p.reshape(2, n1 // 128, 128)
    t1 = _tc_map(lambda a, b: a + b, 128, pr[0], pr[1],
                 block_rows=n1 // 128).reshape(n1)
    q = _spmm1_sc(row, col, val, split, t1)
    h1 = _tc_map(
        lambda xb, t1b, q0b, q1b, W, Ws, bb: jax.nn.relu(
            xb * (W[0][0] + Ws[0] - W[2][0])[None, :]
            + t1b * W[1][0][None, :]
            + (q0b + q1b) * (2.0 * W[2][0])[None, :]
            + bb[0][None, :]),
        16, x, t1[:n, None], q[0, :n, None], q[1, :n, None],
        W1, W1_skip, b1.reshape(1, -1))

    p = spmm(h1)
    T1 = _tc_map(lambda p0, p1: p0 + p1, 16, p[0], p[1])
    q = spmm(T1)
    h2 = _tc_map(
        lambda hb, t1, W, bb, q0, q1: jax.nn.relu(
            _dot(hb, W[0] - W[2]) + _dot(t1, W[1])
            + _dot(q0 + q1, 2.0 * W[2]) + bb[0][None, :]),
        64, h1, T1, W2, b2.reshape(1, -1), q[0], q[1])

    c2 = _tc_map(lambda hb, W: _dot(hb, 2.0 * W[2]), 16, h2, W3)
    p = spmm(c2)
    mid = _tc_map(lambda hb, W, p0, p1: _dot(hb, W[1]) + p0 + p1,
                  16, h2, W3, p[0], p[1])
    q = spmm(mid)
    out = _tc_map(
        lambda hb, W, bb, q0, q1, Wff, bff: _dot(
            jax.nn.relu(_dot(hb, W[0] - W[2]) + q0 + q1 + bb[0][None, :]),
            Wff[0]) + bff[0][None, :],
        1, h2, W3, b3.reshape(1, -1), q[0], q[1], Wf, bf.reshape(1, -1))

    return out[:n_orig]


# speedup vs baseline: 2.3487x; 1.0891x over previous
import functools

import jax
import jax.numpy as jnp
from jax import lax
from jax.experimental import pallas as pl
from jax.experimental.pallas import tpu as pltpu
from jax.experimental.pallas import tpu_sc as plsc

N_TILES = 32
CHUNK = 128
ZROWS = 136
NBUF = 4
HALVES = 2


def _split_chunks(C):
    HPMAX = 125
    n0 = int(round(0.54 * C / 16.0))
    P0 = max(1, -(-n0 // HPMAX))
    n0 -= n0 % P0
    C0 = 16 * n0
    n1 = -(-(C - C0) // 16)
    P1 = max(1, -(-n1 // HPMAX))
    n1 += (-n1) % P1
    return n0, P0, n1, P1, C0 + 16 * n1


def _spmm_sc(row2, col2, val2, split, x):
    n, f = x.shape
    n0, P0, n1, P1, _ = split
    slab = max(n0 // P0, n1 // P1)
    rpt = n // 16
    mesh = plsc.VectorSubcoreMesh(core_axis_name="c", subcore_axis_name="s")

    @functools.partial(
        pl.kernel,
        out_type=jax.ShapeDtypeStruct((2, n, f), jnp.float32),
        mesh=mesh,
        scratch_types=[
            pltpu.VMEM((slab, CHUNK), jnp.int32),
            pltpu.VMEM((slab, CHUNK), jnp.int32),
            pltpu.VMEM((slab, CHUNK), jnp.float32),
            pltpu.VMEM((NBUF, CHUNK, f), jnp.float32),
            pltpu.VMEM((ZROWS, f), jnp.float32),
            pltpu.VMEM_SHARED((n, f), jnp.float32),
        ]
        + [pltpu.SemaphoreType.DMA] * NBUF,
        compiler_params=pltpu.CompilerParams(use_tc_tiling_on_sc=False),
    )
    def spmm_kernel(row_hbm, col_hbm, val_hbm, x_hbm, out_hbm,
                    row_v, col_v, val_v, rows_v, zero_v, acc, *sems):
        c = lax.axis_index("c")
        s = lax.axis_index("s")

        @pl.loop(0, ZROWS)
        def _(r):
            zero_v[r, :] = jnp.zeros((f,), jnp.float32)

        base = s * rpt

        @pl.loop(0, rpt // ZROWS)
        def _(i):
            pltpu.sync_copy(zero_v, acc.at[pl.ds(base + i * ZROWS, ZROWS)])

        plsc.subcore_barrier()

        def run_pass(cstart, hp):
            pltpu.sync_copy(row_hbm.at[pl.ds(cstart, hp)],
                            row_v.at[pl.ds(0, hp)])
            pltpu.sync_copy(col_hbm.at[pl.ds(cstart, hp)],
                            col_v.at[pl.ds(0, hp)])
            pltpu.sync_copy(val_hbm.at[pl.ds(cstart, hp)],
                            val_v.at[pl.ds(0, hp)])

            for b in range(NBUF):
                pltpu.async_copy(x_hbm.at[col_v.at[b]], rows_v.at[b],
                                 sems[b])

            def step(j, b):
                pltpu.make_async_copy(
                    x_hbm.at[col_v.at[j]], rows_v.at[b], sems[b]).wait()

                @pl.loop(0, CHUNK // 16)
                def _(k):
                    vv = val_v[j, pl.ds(k * 16, 16)]
                    for u in range(16):
                        e0 = k * 16 + u
                        rows_v[b, e0, :] = rows_v[b, e0, :] * vv[u]

                pltpu.sync_copy(rows_v.at[b], acc.at[row_v.at[j]],
                                add=True)

            @pl.loop(0, hp // NBUF)
            def _(i):
                for b in range(NBUF):
                    j = i * NBUF + b
                    step(j, b)
                    nxt = j + NBUF

                    @pl.when(nxt < hp)
                    def _():
                        pltpu.async_copy(
                            x_hbm.at[col_v.at[nxt]], rows_v.at[b], sems[b])

            for b in range(hp % NBUF):
                step((hp // NBUF) * NBUF + b, b)

        @pl.when(c == 0)
        def _():
            for p in range(P0):
                run_pass(s * n0 + p * (n0 // P0), n0 // P0)

        @pl.when(c == 1)
        def _():
            for p in range(P1):
                run_pass(16 * n0 + s * n1 + p * (n1 // P1), n1 // P1)

        plsc.subcore_barrier()

        pltpu.sync_copy(acc.at[pl.ds(base, rpt)],
                        out_hbm.at[c, pl.ds(base, rpt)])

    return spmm_kernel(row2, col2, val2, x)


def _spmm1_sc(row2, col2, val2, split, xv):
    n1 = xv.shape[0]
    n0, P0, nc1, P1, _ = split
    slab = max(n0 // P0, nc1 // P1)
    rpt = n1 // 16
    mesh = plsc.VectorSubcoreMesh(core_axis_name="c", subcore_axis_name="s")

    @functools.partial(
        pl.kernel,
        out_type=jax.ShapeDtypeStruct((2, n1), jnp.float32),
        mesh=mesh,
        scratch_types=[
            pltpu.VMEM((slab, CHUNK), jnp.int32),
            pltpu.VMEM((slab, CHUNK), jnp.int32),
            pltpu.VMEM((slab, CHUNK), jnp.float32),
            pltpu.VMEM((NBUF, CHUNK), jnp.float32),
            pltpu.VMEM((rpt,), jnp.float32),
            pltpu.VMEM_SHARED((n1,), jnp.float32),
        ]
        + [pltpu.SemaphoreType.DMA] * NBUF,
        compiler_params=pltpu.CompilerParams(use_tc_tiling_on_sc=False),
    )
    def spmm1_kernel(row_hbm, col_hbm, val_hbm, x_hbm, out_hbm,
                     row_v, col_v, val_v, rows_v, zero_v, acc, *sems):
        c = lax.axis_index("c")
        s = lax.axis_index("s")

        @pl.loop(0, rpt // 16)
        def _(r):
            zero_v[pl.ds(r * 16, 16)] = jnp.zeros((16,), jnp.float32)

        base = s * rpt
        pltpu.sync_copy(zero_v, acc.at[pl.ds(base, rpt)])
        plsc.subcore_barrier()

        def run_pass(cstart, hp):
            pltpu.sync_copy(row_hbm.at[pl.ds(cstart, hp)],
                            row_v.at[pl.ds(0, hp)])
            pltpu.sync_copy(col_hbm.at[pl.ds(cstart, hp)],
                            col_v.at[pl.ds(0, hp)])
            pltpu.sync_copy(val_hbm.at[pl.ds(cstart, hp)],
                            val_v.at[pl.ds(0, hp)])

            for b in range(NBUF):
                pltpu.async_copy(x_hbm.at[col_v.at[b]], rows_v.at[b],
                                 sems[b])

            def step(j, b):
                pltpu.make_async_copy(
                    x_hbm.at[col_v.at[j]], rows_v.at[b], sems[b]).wait()

                @pl.loop(0, CHUNK // 16)
                def _(k):
                    sl = pl.ds(k * 16, 16)
                    rows_v[b, sl] = rows_v[b, sl] * val_v[j, sl]

                pltpu.sync_copy(rows_v.at[b], acc.at[row_v.at[j]],
                                add=True)

            @pl.loop(0, hp // NBUF)
            def _(i):
                for b in range(NBUF):
                    j = i * NBUF + b
                    step(j, b)
                    nxt = j + NBUF

                    @pl.when(nxt < hp)
                    def _():
                        pltpu.async_copy(
                            x_hbm.at[col_v.at[nxt]], rows_v.at[b], sems[b])

            for b in range(hp % NBUF):
                step((hp // NBUF) * NBUF + b, b)

        @pl.when(c == 0)
        def _():
            for p in range(P0):
                run_pass(s * n0 + p * (n0 // P0), n0 // P0)

        @pl.when(c == 1)
        def _():
            for p in range(P1):
                run_pass(16 * n0 + s * nc1 + p * (nc1 // P1), nc1 // P1)

        plsc.subcore_barrier()

        pltpu.sync_copy(acc.at[pl.ds(base, rpt)],
                        out_hbm.at[c, pl.ds(base, rpt)])

    return spmm1_kernel(row2, col2, val2, xv)


def _tc_map(body, out_feat, *arrays, block_rows=6256):
    n = arrays[0].shape[0]
    grid = (n // block_rows,)
    in_specs = []
    for a in arrays:
        if a.shape[0] == n:
            blk = (block_rows,) + a.shape[1:]
            in_specs.append(pl.BlockSpec(blk, lambda i, _nd=a.ndim: (i,) + (0,) * (_nd - 1)))
        else:
            in_specs.append(pl.BlockSpec(a.shape, lambda i, _nd=a.ndim: (0,) * _nd))
    out = jax.ShapeDtypeStruct((n, out_feat), jnp.float32)

    def wrapped(*refs):
        ins = refs[:-1]
        o = refs[-1]
        o[...] = body(*[r[...] for r in ins])

    return pl.pallas_call(
        wrapped,
        grid=grid,
        in_specs=in_specs,
        out_specs=pl.BlockSpec((block_rows, out_feat), lambda i: (i, 0)),
        out_shape=out,
    )(*arrays)


def _dot(a, b):
    return jax.lax.dot_general(a, b, (((1,), (0,)), ((), ())),
                               preferred_element_type=jnp.float32)


def kernel(laplacian_indices, laplacian_values, x,
           W1, W1_skip, b1, W2, b2, W3, b3, Wf, bf):
    n_orig = x.shape[0]
    e = laplacian_values.shape[0]

    n = -(-n_orig // (16 * ZROWS)) * (16 * ZROWS)
    x = jnp.pad(x, ((0, n - n_orig), (0, 0)))

    split = _split_chunks(-(-e // CHUNK))
    e_pad = split[4] * CHUNK
    pad = e_pad - e
    row = jnp.pad(laplacian_indices[0], (0, pad)).reshape(-1, CHUNK)
    col = jnp.pad(laplacian_indices[1], (0, pad)).reshape(-1, CHUNK)
    val = jnp.pad(laplacian_values, (0, pad)).reshape(-1, CHUNK)

    spmm = lambda h: _spmm_sc(row, col, val, split, h)

    n1 = -(-n_orig // 256) * 256
    x1 = jnp.pad(x[:, 0], (0, n1 - n))
    p = _spmm1_sc(row, col, val, split, x1)
    pr = p.reshape(2, n1 // 128, 128)
    t1 = _tc_map(lambda a, b: a + b, 128, pr[0], pr[1],
                 block_rows=n1 // 128).reshape(n1)
    q = _spmm1_sc(row, col, val, split, t1)
    h1 = _tc_map(
        lambda xb, t1b, q0b, q1b, W, Ws, bb: jax.nn.relu(
            xb * (W[0][0] + Ws[0] - W[2][0])[None, :]
            + t1b * W[1][0][None, :]
            + (q0b + q1b) * (2.0 * W[2][0])[None, :]
            + bb[0][None, :]),
        16, x, t1[:n, None], q[0, :n, None], q[1, :n, None],
        W1, W1_skip, b1.reshape(1, -1))

    p = spmm(h1)
    T1 = _tc_map(lambda p0, p1: p0 + p1, 16, p[0], p[1])
    q = spmm(T1)
    h2 = _tc_map(
        lambda hb, t1, W, bb, q0, q1: jax.nn.relu(
            _dot(hb, W[0] - W[2]) + _dot(t1, W[1])
            + _dot(q0 + q1, 2.0 * W[2]) + bb[0][None, :]),
        64, h1, T1, W2, b2.reshape(1, -1), q[0], q[1])

    c2 = _tc_map(lambda hb, W: _dot(hb, 2.0 * W[2]), 16, h2, W3)
    p = spmm(c2)
    mid = _tc_map(lambda hb, W, p0, p1: _dot(hb, W[1]) + p0 + p1,
                  16, h2, W3, p[0], p[1])
    q = spmm(mid)
    out = _tc_map(
        lambda hb, W, bb, q0, q1, Wff, bff: _dot(
            jax.nn.relu(_dot(hb, W[0] - W[2]) + q0 + q1 + bb[0][None, :]),
            Wff[0]) + bff[0][None, :],
        1, h2, W3, b3.reshape(1, -1), q[0], q[1], Wf, bf.reshape(1, -1))

    return out[:n_orig]
